# Initial kernel scaffold; baseline (speedup 1.0000x reference)
import jax
import jax.numpy as jnp
from jax.experimental import pallas as pl


def kernel(logits, hidden_state, prev_hidden, generated_ids, W, bias):
    raise NotImplementedError("write your pallas kernel here")


# speedup vs baseline: 2.1597x; 2.1597x over previous
import functools

import jax
import jax.numpy as jnp
import numpy as np
from jax import lax
from jax.experimental import pallas as pl
from jax.experimental.pallas import tpu as pltpu
from jax.experimental.pallas import tpu_sc as plsc

PW = 0.1
NGRAM = 4


def _sim_body(h_ref, p_ref, w_ref, out_ref):
    dn = (((1,), (1,)), ((), ()))
    h1 = lax.dot_general(h_ref[...], w_ref[...], dn,
                         preferred_element_type=jnp.float32)
    h2 = lax.dot_general(p_ref[...], w_ref[...], dn,
                         preferred_element_type=jnp.float32)
    dot = jnp.sum(h1 * h2, axis=-1)
    n1 = jnp.maximum(jnp.sqrt(jnp.sum(h1 * h1, axis=-1)), 1e-8)
    n2 = jnp.maximum(jnp.sqrt(jnp.sum(h2 * h2, axis=-1)), 1e-8)
    sim = dot / (n1 * n2)
    out_ref[...] = (jnp.clip(sim, 0.0, None) * PW)[None, :]


def _sim_call(hidden, prev, w):
    b = hidden.shape[0]
    return pl.pallas_call(
        _sim_body,
        out_shape=jax.ShapeDtypeStruct((1, b), jnp.float32),
    )(hidden, prev, w)


def _ngram_body(t0_ref, t1_ref, t2_ref, t3_ref, sim_ref, temp_ref, *, wn):
    a0, a1 = t0_ref[...], t1_ref[...]
    a2, a3 = t2_ref[...], t3_ref[...]
    minint = jnp.int32(-(2 ** 31))
    k1 = ((a0 << 15) | (a1 >> 2)) ^ minint
    k2 = ((a1 & 3) << 17) | a2
    k3 = a3

    rows, cols = k1.shape
    half = cols // 2
    row_i = lax.broadcasted_iota(jnp.int32, (rows, cols), 0)
    lane_i = lax.broadcasted_iota(jnp.int32, (rows, cols), 1)
    ivirt = ((lane_i // half) * rows) | row_i

    def stage(arrs, j, k):
        b1, b2, b3 = arrs
        d1 = pltpu.roll(b1, rows - j, 0)
        d2 = pltpu.roll(b2, rows - j, 0)
        d3 = pltpu.roll(b3, rows - j, 0)
        u1 = pltpu.roll(b1, j, 0)
        u2 = pltpu.roll(b2, j, 0)
        u3 = pltpu.roll(b3, j, 0)
        mine_low = (row_i & j) == 0
        p1 = jnp.where(mine_low, d1, u1)
        p2 = jnp.where(mine_low, d2, u2)
        p3 = jnp.where(mine_low, d3, u3)
        asc = (ivirt & k) == 0
        less = (b1 < p1) | ((b1 == p1) & ((b2 < p2) | ((b2 == p2) & (b3 < p3))))
        sel = (mine_low == asc) == less
        return (jnp.where(sel, b1, p1),
                jnp.where(sel, b2, p2),
                jnp.where(sel, b3, p3))

    def jloop(arrs, ke, js_exp, nj):
        def body(m, a):
            return stage(a, jnp.int32(1) << (js_exp - m), jnp.int32(1) << ke)
        return lax.fori_loop(0, nj, body, arrs)

    arrs = (k1, k2, k3)
    lg = rows.bit_length() - 1
    arrs = lax.fori_loop(
        1, lg + 1, lambda kk, a: jloop(a, kk, kk - 1, kk), arrs)
    b1, b2, b3 = arrs
    p1 = pltpu.roll(b1, half, 1)
    p2 = pltpu.roll(b2, half, 1)
    p3 = pltpu.roll(b3, half, 1)
    mine_low = (lane_i & half) == 0
    less = (b1 < p1) | ((b1 == p1) & ((b2 < p2) | ((b2 == p2) & (b3 < p3))))
    sel = mine_low == less
    arrs = (jnp.where(sel, b1, p1), jnp.where(sel, b2, p2),
            jnp.where(sel, b3, p3))
    arrs = jloop(arrs, lg + 1, lg - 1, lg)

    s1, s2, s3 = arrs
    neq = ((s1[1:] != s1[:-1]) | (s2[1:] != s2[:-1])
           | (s3[1:] != s3[:-1]))
    csum = jnp.sum(neq.astype(jnp.int32), axis=0, keepdims=True)
    tops = (pltpu.roll(s1[:1], half, 1), pltpu.roll(s2[:1], half, 1),
            pltpu.roll(s3[:1], half, 1))
    bnd = ((s1[rows - 1:] != tops[0]) | (s2[rows - 1:] != tops[1])
           | (s3[rows - 1:] != tops[2])).astype(jnp.int32)
    uniq = (csum[:, :half] + csum[:, half:] + bnd[:, :half]).astype(jnp.float32)
    rep = (wn - uniq) / wn
    temp_ref[...] = 1.0 + (rep + sim_ref[...]) * 0.5


def _ngram_call(tks, sim, wn):
    b2 = tks[0].shape[1]
    return pl.pallas_call(
        functools.partial(_ngram_body, wn=float(wn)),
        out_shape=jax.ShapeDtypeStruct((1, b2 // 2), jnp.float32),
    )(*tks, sim)


def _bias_body(l_ref, b_ref, o_ref):
    o_ref[...] = l_ref[...] + b_ref[...]


def _bias_call(logits, bias):
    b, v = logits.shape
    blk = 12800
    grid = (v + blk - 1) // blk
    return pl.pallas_call(
        _bias_body,
        grid=(grid,),
        in_specs=[pl.BlockSpec((b, blk), lambda i: (0, i)),
                  pl.BlockSpec((1, blk), lambda i: (0, i))],
        out_specs=pl.BlockSpec((b, blk), lambda i: (0, i)),
        out_shape=jax.ShapeDtypeStruct((b, v), jnp.float32),
    )(logits, bias.reshape(1, v))


def _freq_call(out0, gen):
    b, v = out0.shape
    s = gen.shape[1]
    lanes = 16
    nw = 32
    rows_per_w = b // nw
    c = float(np.float32(PW) / np.float32(s))
    mesh = plsc.VectorSubcoreMesh(core_axis_name="c", subcore_axis_name="s")

    @functools.partial(
        pl.kernel,
        out_type=jax.ShapeDtypeStruct((b, v), jnp.float32),
        mesh=mesh,
        compiler_params=pltpu.CompilerParams(needs_layout_passes=False),
        scratch_types=[
            pltpu.VMEM((v,), jnp.float32),
            pltpu.VMEM((s,), jnp.int32),
        ],
    )
    def freq_kernel(out0_hbm, gen_hbm, out_hbm, buf, tok):
        wid = lax.axis_index("s") * 2 + lax.axis_index("c")
        for r in range(rows_per_w):
            row = wid * rows_per_w + r
            pltpu.sync_copy(out0_hbm.at[row], buf)
            pltpu.sync_copy(gen_hbm.at[row], tok)

            def body(i, carry):
                idx = tok[pl.ds(i * lanes, lanes)]
                cnt, last = plsc.scan_count(idx)
                plsc.addupdate_scatter(
                    buf, [idx], cnt.astype(jnp.float32) * (-c), mask=last)
                return carry

            lax.fori_loop(0, s // lanes, body, 0)
            pltpu.sync_copy(buf, out_hbm.at[row])

    return freq_kernel(out0, gen)


def kernel(logits, hidden_state, prev_hidden, generated_ids, W, bias):
    b, v = logits.shape
    s = generated_ids.shape[1]
    wn = s - NGRAM + 1

    sim = _sim_call(hidden_state, prev_hidden, W)

    g_t = generated_ids.T
    pad_vals = (v, 0, 0, 0)
    tks = []
    for k in range(NGRAM):
        tk = jnp.concatenate(
            [g_t[k:k + wn],
             jnp.full((s - wn, b), pad_vals[k], jnp.int32)], axis=0)
        tk = tk.reshape(2, s // 2, b).transpose(1, 0, 2).reshape(s // 2, 2 * b)
        tks.append(tk)
    temp = _ngram_call(tks, sim, wn)

    out0 = _bias_call(logits, bias)
    out = _freq_call(out0, generated_ids)
    return out, temp.reshape(b)


# speedup vs baseline: 4.3939x; 2.0345x over previous
import functools

import jax
import jax.numpy as jnp
import numpy as np
from jax import lax
from jax.experimental import pallas as pl
from jax.experimental.pallas import tpu as pltpu
from jax.experimental.pallas import tpu_sc as plsc

PW = 0.1
NGRAM = 4
_SC_PARAMS = pltpu.CompilerParams(needs_layout_passes=False)


def _ngram_call(gen, v):
    b, s = gen.shape
    wn = s - NGRAM + 1
    nv = s // 16
    padw = s
    sent = s
    nbins = s + 1
    histw = ((nbins + 15) // 16) * 16
    rows_per_w = b // 32
    mesh = plsc.VectorSubcoreMesh(core_axis_name="c", subcore_axis_name="s")

    @functools.partial(
        pl.kernel,
        out_type=jax.ShapeDtypeStruct((b, 16), jnp.int32),
        mesh=mesh,
        compiler_params=_SC_PARAMS,
        scratch_types=[
            pltpu.VMEM((v,), jnp.int32),
            pltpu.VMEM((s,), jnp.int32),
            pltpu.VMEM((s + 16,), jnp.int32),
            pltpu.VMEM((s + 16,), jnp.int32),
            pltpu.VMEM((s + 16,), jnp.int32),
            pltpu.VMEM((histw,), jnp.int32),
            pltpu.VMEM((16,), jnp.int32),
        ],
    )
    def ngram_kernel(gen_hbm, cnt_hbm, table, tok, tids, ws_a, ws_b, hist,
                     out16):
        wid = lax.axis_index("s") * 2 + lax.axis_index("c")
        lanes = lax.broadcasted_iota(jnp.int32, (16,), 0)
        for r in range(rows_per_w):
            row = wid * rows_per_w + r
            pltpu.sync_copy(gen_hbm.at[row], tok)

            def build_table(i, c):
                t = tok[pl.ds(i * 16, 16)]
                _, last = plsc.scan_count(t)
                plsc.store_scatter(table, [t], i * 16 + lanes, mask=last)
                return c
            lax.fori_loop(0, nv, build_table, 0)

            def fill_ids(i, c):
                t = tok[pl.ds(i * 16, 16)]
                tids[pl.ds(i * 16, 16)] = plsc.load_gather(table, [t])
                return c
            lax.fori_loop(0, nv, fill_ids, 0)
            tids[pl.ds(s, 16)] = jnp.full((16,), sent, jnp.int32)

            def init_order(i, c):
                vals = i * 16 + lanes
                ws_a[pl.ds(i * 16, 16)] = jnp.where(vals < wn, vals, padw)
                return c
            lax.fori_loop(0, nv, init_order, 0)

            for p in range(NGRAM):
                src, dst = (ws_a, ws_b) if p % 2 == 0 else (ws_b, ws_a)
                koff = NGRAM - 1 - p

                def zero_hist(i, c):
                    hist[pl.ds(i * 16, 16)] = jnp.zeros((16,), jnp.int32)
                    return c
                lax.fori_loop(0, histw // 16, zero_hist, 0)

                def histo(i, c):
                    w = src[pl.ds(i * 16, 16)]
                    d = plsc.load_gather(tids, [w + koff])
                    occ, last = plsc.scan_count(d)
                    plsc.addupdate_scatter(hist, [d], occ, mask=last)
                    return c
                lax.fori_loop(0, nv, histo, 0)

                def excl_scan(i, run):
                    h = hist[pl.ds(i * 16, 16)]
                    csum = plsc.cumsum(h)
                    hist[pl.ds(i * 16, 16)] = csum - h + run
                    return run + jnp.sum(h)
                lax.fori_loop(0, histw // 16, excl_scan, jnp.int32(0))

                def permute(i, c):
                    w = src[pl.ds(i * 16, 16)]
                    d = plsc.load_gather(tids, [w + koff])
                    base = plsc.load_gather(hist, [d])
                    occ, last = plsc.scan_count(d)
                    plsc.store_scatter(dst, [base + occ - 1], w)
                    plsc.addupdate_scatter(hist, [d], occ, mask=last)
                    return c
                lax.fori_loop(0, nv, permute, 0)

            final = ws_a if NGRAM % 2 == 0 else ws_b
            final[pl.ds(s, 16)] = jnp.full((16,), padw, jnp.int32)

            def count(i, acc):
                a = final[pl.ds(i * 16, 16)]
                bb = final[pl.ds(i * 16 + 1, 16)]
                neq = jnp.zeros((16,), jnp.bool_)
                for k in range(NGRAM):
                    da = plsc.load_gather(tids, [a + k])
                    db = plsc.load_gather(tids, [bb + k])
                    neq = neq | (da != db)
                return acc + plsc.all_reduce_population_count(neq)
            acc = lax.fori_loop(0, nv, count,
                                jnp.zeros((16,), jnp.int32))
            out16[...] = acc
            pltpu.sync_copy(out16, cnt_hbm.at[row])

    return ngram_kernel(gen)


def _sim_temp_body(h_ref, p_ref, w_ref, cnt_ref, out_ref, *, wn):
    dn = (((1,), (1,)), ((), ()))
    h1 = lax.dot_general(h_ref[...], w_ref[...], dn,
                         preferred_element_type=jnp.float32)
    h2 = lax.dot_general(p_ref[...], w_ref[...], dn,
                         preferred_element_type=jnp.float32)
    dot = jnp.sum(h1 * h2, axis=-1)
    n1 = jnp.maximum(jnp.sqrt(jnp.sum(h1 * h1, axis=-1)), 1e-8)
    n2 = jnp.maximum(jnp.sqrt(jnp.sum(h2 * h2, axis=-1)), 1e-8)
    sim_pen = jnp.clip(dot / (n1 * n2), 0.0, None) * PW
    uniq = cnt_ref[...][:, 0].astype(jnp.float32)
    rep = (wn - uniq) / wn
    out_ref[...] = (1.0 + (rep + sim_pen) * 0.5)[None, :]


def _sim_temp_call(hidden, prev, w, cnts, wn):
    b = hidden.shape[0]
    return pl.pallas_call(
        functools.partial(_sim_temp_body, wn=float(wn)),
        out_shape=jax.ShapeDtypeStruct((1, b), jnp.float32),
    )(hidden, prev, w, cnts)


def _bias_body(l_ref, b_ref, o_ref):
    o_ref[...] = l_ref[...] + b_ref[...]


def _bias_call(logits, bias):
    b, v = logits.shape
    blk = 12800
    grid = (v + blk - 1) // blk
    return pl.pallas_call(
        _bias_body,
        grid=(grid,),
        in_specs=[pl.BlockSpec((b, blk), lambda i: (0, i)),
                  pl.BlockSpec((1, blk), lambda i: (0, i))],
        out_specs=pl.BlockSpec((b, blk), lambda i: (0, i)),
        out_shape=jax.ShapeDtypeStruct((b, v), jnp.float32),
    )(logits, bias.reshape(1, v))


def _freq_call(out0, gen):
    b, v = out0.shape
    s = gen.shape[1]
    lanes = 16
    rows_per_w = b // 32
    c = float(np.float32(PW) / np.float32(s))
    mesh = plsc.VectorSubcoreMesh(core_axis_name="c", subcore_axis_name="s")

    @functools.partial(
        pl.kernel,
        out_type=jax.ShapeDtypeStruct((b, v), jnp.float32),
        mesh=mesh,
        compiler_params=_SC_PARAMS,
        scratch_types=[
            pltpu.VMEM((v,), jnp.float32),
            pltpu.VMEM((s,), jnp.int32),
        ],
    )
    def freq_kernel(out0_hbm, gen_hbm, out_hbm, buf, tok):
        wid = lax.axis_index("s") * 2 + lax.axis_index("c")
        for r in range(rows_per_w):
            row = wid * rows_per_w + r
            pltpu.sync_copy(out0_hbm.at[row], buf)
            pltpu.sync_copy(gen_hbm.at[row], tok)

            def body(i, carry):
                idx = tok[pl.ds(i * lanes, lanes)]
                cnt, last = plsc.scan_count(idx)
                plsc.addupdate_scatter(
                    buf, [idx], cnt.astype(jnp.float32) * (-c), mask=last)
                return carry

            lax.fori_loop(0, s // lanes, body, 0)
            pltpu.sync_copy(buf, out_hbm.at[row])

    return freq_kernel(out0, gen)


def kernel(logits, hidden_state, prev_hidden, generated_ids, W, bias):
    b, v = logits.shape
    s = generated_ids.shape[1]
    wn = s - NGRAM + 1

    cnts = _ngram_call(generated_ids, v)
    out0 = _bias_call(logits, bias)
    out = _freq_call(out0, generated_ids)
    temp = _sim_temp_call(hidden_state, prev_hidden, W, cnts, wn)
    return out, temp.reshape(b)


# speedup vs baseline: 5.8900x; 1.3405x over previous
import functools

import jax
import jax.numpy as jnp
import numpy as np
from jax import lax
from jax.experimental import pallas as pl
from jax.experimental.pallas import tpu as pltpu
from jax.experimental.pallas import tpu_sc as plsc

PW = 0.1
NGRAM = 4
_SC_PARAMS = pltpu.CompilerParams(needs_layout_passes=False)


def _ngram_call(gen, v):
    b, s = gen.shape
    wn = s - NGRAM + 1
    nv = s // 16
    padw = s
    sent = s
    nbins = s + 1
    histw = ((nbins + 15) // 16) * 16
    rows_per_w = b // 32
    mesh = plsc.VectorSubcoreMesh(core_axis_name="c", subcore_axis_name="s")

    @functools.partial(
        pl.kernel,
        out_type=jax.ShapeDtypeStruct((b, 16), jnp.int32),
        mesh=mesh,
        compiler_params=_SC_PARAMS,
        scratch_types=[
            pltpu.VMEM((v,), jnp.int32),
            pltpu.VMEM((s,), jnp.int32),
            pltpu.VMEM((s + 16,), jnp.int32),
            pltpu.VMEM((s + 16,), jnp.int32),
            pltpu.VMEM((s + 16,), jnp.int32),
            pltpu.VMEM((s + 16,), jnp.int32),
            pltpu.VMEM((histw,), jnp.int32),
            pltpu.VMEM((16,), jnp.int32),
        ],
    )
    def ngram_kernel(gen_hbm, cnt_hbm, table, tok, tids, dpos, ws_a, ws_b,
                     hist, out16):
        wid = lax.axis_index("s") * 2 + lax.axis_index("c")
        lanes = lax.broadcasted_iota(jnp.int32, (16,), 0)

        def zero_table(i, c):
            table[pl.ds(i * 16, 16)] = jnp.zeros((16,), jnp.int32)
            return c
        lax.fori_loop(0, v // 16, zero_table, 0)
        dpos[pl.ds(s, 16)] = jnp.zeros((16,), jnp.int32)

        def radix_distinct():
            def build_table(i, c):
                t = tok[pl.ds(i * 16, 16)]
                _, last = plsc.scan_count(t)
                plsc.store_scatter(table, [t], i * 16 + lanes, mask=last)
                return c
            lax.fori_loop(0, nv, build_table, 0)

            def fill_ids(i, c):
                t = tok[pl.ds(i * 16, 16)]
                tids[pl.ds(i * 16, 16)] = plsc.load_gather(table, [t])
                return c
            lax.fori_loop(0, nv, fill_ids, 0)
            tids[pl.ds(s, 16)] = jnp.full((16,), sent, jnp.int32)

            def init_order(i, c):
                vals = i * 16 + lanes
                ws_a[pl.ds(i * 16, 16)] = jnp.where(vals < wn, vals, padw)
                return c
            lax.fori_loop(0, nv, init_order, 0)

            for p in range(NGRAM):
                src, dst = (ws_a, ws_b) if p % 2 == 0 else (ws_b, ws_a)
                koff = NGRAM - 1 - p

                def zero_hist(i, c):
                    hist[pl.ds(i * 16, 16)] = jnp.zeros((16,), jnp.int32)
                    return c
                lax.fori_loop(0, histw // 16, zero_hist, 0)

                def histo(i, c):
                    w = src[pl.ds(i * 16, 16)]
                    d = plsc.load_gather(tids, [w + koff])
                    occ, last = plsc.scan_count(d)
                    plsc.addupdate_scatter(hist, [d], occ, mask=last)
                    return c
                lax.fori_loop(0, nv, histo, 0)

                def excl_scan(i, run):
                    h = hist[pl.ds(i * 16, 16)]
                    csum = plsc.cumsum(h)
                    hist[pl.ds(i * 16, 16)] = csum - h + run
                    return run + jnp.sum(h)
                lax.fori_loop(0, histw // 16, excl_scan, jnp.int32(0))

                def permute(i, c):
                    w = src[pl.ds(i * 16, 16)]
                    d = plsc.load_gather(tids, [w + koff])
                    base = plsc.load_gather(hist, [d])
                    occ, last = plsc.scan_count(d)
                    plsc.store_scatter(dst, [base + occ - 1], w)
                    plsc.addupdate_scatter(hist, [d], occ, mask=last)
                    return c
                lax.fori_loop(0, nv, permute, 0)

            final = ws_a if NGRAM % 2 == 0 else ws_b
            final[pl.ds(s, 16)] = jnp.full((16,), padw, jnp.int32)

            def count(i, acc):
                a = final[pl.ds(i * 16, 16)]
                bb = final[pl.ds(i * 16 + 1, 16)]
                neq = jnp.zeros((16,), jnp.bool_)
                for k in range(NGRAM):
                    da = plsc.load_gather(tids, [a + k])
                    db = plsc.load_gather(tids, [bb + k])
                    neq = neq | (da != db)
                return acc + plsc.all_reduce_population_count(neq)
            return lax.fori_loop(0, nv, count, jnp.zeros((16,), jnp.int32))

        for r in range(rows_per_w):
            row = wid * rows_per_w + r
            pltpu.sync_copy(gen_hbm.at[row], tok)

            def count_toks(i, c):
                t = tok[pl.ds(i * 16, 16)]
                occ, last = plsc.scan_count(t)
                plsc.addupdate_scatter(table, [t], occ, mask=last)
                return c
            lax.fori_loop(0, nv, count_toks, 0)

            def dup_flags(i, c):
                t = tok[pl.ds(i * 16, 16)]
                cnt = plsc.load_gather(table, [t])
                dpos[pl.ds(i * 16, 16)] = (cnt >= 2).astype(jnp.int32)
                return c
            lax.fori_loop(0, nv, dup_flags, 0)

            def count_cands(i, ncand):
                f = dpos[pl.ds(i * 16, 16)]
                for k in range(1, NGRAM):
                    f = f & dpos[pl.ds(i * 16 + k, 16)]
                valid = (i * 16 + lanes) < wn
                return ncand + jnp.sum(jnp.where(valid, f, 0))
            ncand = lax.fori_loop(0, nv, count_cands, jnp.int32(0))

            acc = lax.cond(
                ncand == 0,
                lambda: jnp.full((16,), wn, jnp.int32),
                radix_distinct)

            def rezero(i, c):
                t = tok[pl.ds(i * 16, 16)]
                plsc.store_scatter(table, [t], jnp.zeros((16,), jnp.int32))
                return c
            lax.fori_loop(0, nv, rezero, 0)

            out16[...] = acc
            pltpu.sync_copy(out16, cnt_hbm.at[row])

    return ngram_kernel(gen)


def _sim_temp_body(h_ref, p_ref, w_ref, cnt_ref, out_ref, *, wn):
    dn = (((1,), (1,)), ((), ()))
    h1 = lax.dot_general(h_ref[...], w_ref[...], dn,
                         preferred_element_type=jnp.float32)
    h2 = lax.dot_general(p_ref[...], w_ref[...], dn,
                         preferred_element_type=jnp.float32)
    dot = jnp.sum(h1 * h2, axis=-1)
    n1 = jnp.maximum(jnp.sqrt(jnp.sum(h1 * h1, axis=-1)), 1e-8)
    n2 = jnp.maximum(jnp.sqrt(jnp.sum(h2 * h2, axis=-1)), 1e-8)
    sim_pen = jnp.clip(dot / (n1 * n2), 0.0, None) * PW
    uniq = cnt_ref[...][:, 0].astype(jnp.float32)
    rep = (wn - uniq) / wn
    out_ref[...] = (1.0 + (rep + sim_pen) * 0.5)[None, :]


def _sim_temp_call(hidden, prev, w, cnts, wn):
    b = hidden.shape[0]
    return pl.pallas_call(
        functools.partial(_sim_temp_body, wn=float(wn)),
        out_shape=jax.ShapeDtypeStruct((1, b), jnp.float32),
    )(hidden, prev, w, cnts)


def _bias_body(l_ref, b_ref, o_ref):
    o_ref[...] = l_ref[...] + b_ref[...]


def _bias_call(logits, bias):
    b, v = logits.shape
    blk = 12800
    grid = (v + blk - 1) // blk
    return pl.pallas_call(
        _bias_body,
        grid=(grid,),
        in_specs=[pl.BlockSpec((b, blk), lambda i: (0, i)),
                  pl.BlockSpec((1, blk), lambda i: (0, i))],
        out_specs=pl.BlockSpec((b, blk), lambda i: (0, i)),
        out_shape=jax.ShapeDtypeStruct((b, v), jnp.float32),
    )(logits, bias.reshape(1, v))


def _freq_call(out0, gen):
    b, v = out0.shape
    s = gen.shape[1]
    lanes = 16
    rows_per_w = b // 32
    c = float(np.float32(PW) / np.float32(s))
    mesh = plsc.VectorSubcoreMesh(core_axis_name="c", subcore_axis_name="s")

    @functools.partial(
        pl.kernel,
        out_type=jax.ShapeDtypeStruct((b, v), jnp.float32),
        mesh=mesh,
        compiler_params=_SC_PARAMS,
        scratch_types=[
            pltpu.VMEM((v,), jnp.float32),
            pltpu.VMEM((s,), jnp.int32),
        ],
    )
    def freq_kernel(out0_hbm, gen_hbm, out_hbm, buf, tok):
        wid = lax.axis_index("s") * 2 + lax.axis_index("c")
        for r in range(rows_per_w):
            row = wid * rows_per_w + r
            pltpu.sync_copy(out0_hbm.at[row], buf)
            pltpu.sync_copy(gen_hbm.at[row], tok)

            def body(i, carry):
                idx = tok[pl.ds(i * lanes, lanes)]
                cnt, last = plsc.scan_count(idx)
                plsc.addupdate_scatter(
                    buf, [idx], cnt.astype(jnp.float32) * (-c), mask=last)
                return carry

            lax.fori_loop(0, s // lanes, body, 0)
            pltpu.sync_copy(buf, out_hbm.at[row])

    return freq_kernel(out0, gen)


def kernel(logits, hidden_state, prev_hidden, generated_ids, W, bias):
    b, v = logits.shape
    s = generated_ids.shape[1]
    wn = s - NGRAM + 1

    cnts = _ngram_call(generated_ids, v)
    out0 = _bias_call(logits, bias)
    out = _freq_call(out0, generated_ids)
    temp = _sim_temp_call(hidden_state, prev_hidden, W, cnts, wn)
    return out, temp.reshape(b)


# speedup vs baseline: 7.6156x; 1.2930x over previous
import functools

import jax
import jax.numpy as jnp
import numpy as np
from jax import lax
from jax.experimental import pallas as pl
from jax.experimental.pallas import tpu as pltpu
from jax.experimental.pallas import tpu_sc as plsc

PW = 0.1
NGRAM = 4
_SC_PARAMS = pltpu.CompilerParams(needs_layout_passes=False)


def _ngram_call(gen, v):
    b, s = gen.shape
    wn = s - NGRAM + 1
    nv = s // 16
    padw = s
    sent = s
    nbins = s + 1
    histw = ((nbins + 15) // 16) * 16
    rows_per_w = b // 32
    mesh = plsc.VectorSubcoreMesh(core_axis_name="c", subcore_axis_name="s")

    @functools.partial(
        pl.kernel,
        out_type=jax.ShapeDtypeStruct((b, 16), jnp.int32),
        mesh=mesh,
        compiler_params=_SC_PARAMS,
        scratch_types=[
            pltpu.VMEM((v,), jnp.int32),
            pltpu.VMEM((s,), jnp.int32),
            pltpu.VMEM((s + 16,), jnp.int32),
            pltpu.VMEM((s + 16,), jnp.int32),
            pltpu.VMEM((s + 16,), jnp.int32),
            pltpu.VMEM((s + 16,), jnp.int32),
            pltpu.VMEM((histw,), jnp.int32),
            pltpu.VMEM((16,), jnp.int32),
        ],
    )
    def ngram_kernel(gen_hbm, cnt_hbm, table, tok, tids, dpos, ws_a, ws_b,
                     hist, out16):
        wid = lax.axis_index("s") * 2 + lax.axis_index("c")
        lanes = lax.broadcasted_iota(jnp.int32, (16,), 0)
        dpos[pl.ds(s, 16)] = jnp.zeros((16,), jnp.int32)

        def radix_distinct():
            def build_table(i, c):
                t = tok[pl.ds(i * 16, 16)]
                _, last = plsc.scan_count(t)
                plsc.store_scatter(table, [t], i * 16 + lanes, mask=last)
                return c
            lax.fori_loop(0, nv, build_table, 0)

            def fill_ids(i, c):
                t = tok[pl.ds(i * 16, 16)]
                tids[pl.ds(i * 16, 16)] = plsc.load_gather(table, [t])
                return c
            lax.fori_loop(0, nv, fill_ids, 0)
            tids[pl.ds(s, 16)] = jnp.full((16,), sent, jnp.int32)

            def init_order(i, c):
                vals = i * 16 + lanes
                ws_a[pl.ds(i * 16, 16)] = jnp.where(vals < wn, vals, padw)
                return c
            lax.fori_loop(0, nv, init_order, 0)

            for p in range(NGRAM):
                src, dst = (ws_a, ws_b) if p % 2 == 0 else (ws_b, ws_a)
                koff = NGRAM - 1 - p

                def zero_hist(i, c):
                    hist[pl.ds(i * 16, 16)] = jnp.zeros((16,), jnp.int32)
                    return c
                lax.fori_loop(0, histw // 16, zero_hist, 0)

                def histo(i, c):
                    w = src[pl.ds(i * 16, 16)]
                    d = plsc.load_gather(tids, [w + koff])
                    occ, last = plsc.scan_count(d)
                    plsc.addupdate_scatter(hist, [d], occ, mask=last)
                    return c
                lax.fori_loop(0, nv, histo, 0)

                def excl_scan(i, run):
                    h = hist[pl.ds(i * 16, 16)]
                    csum = plsc.cumsum(h)
                    hist[pl.ds(i * 16, 16)] = csum - h + run
                    return run + jnp.sum(h)
                lax.fori_loop(0, histw // 16, excl_scan, jnp.int32(0))

                def permute(i, c):
                    w = src[pl.ds(i * 16, 16)]
                    d = plsc.load_gather(tids, [w + koff])
                    base = plsc.load_gather(hist, [d])
                    occ, last = plsc.scan_count(d)
                    plsc.store_scatter(dst, [base + occ - 1], w)
                    plsc.addupdate_scatter(hist, [d], occ, mask=last)
                    return c
                lax.fori_loop(0, nv, permute, 0)

            final = ws_a if NGRAM % 2 == 0 else ws_b
            final[pl.ds(s, 16)] = jnp.full((16,), padw, jnp.int32)

            def count(i, acc):
                a = final[pl.ds(i * 16, 16)]
                bb = final[pl.ds(i * 16 + 1, 16)]
                neq = jnp.zeros((16,), jnp.bool_)
                for k in range(NGRAM):
                    da = plsc.load_gather(tids, [a + k])
                    db = plsc.load_gather(tids, [bb + k])
                    neq = neq | (da != db)
                return acc + plsc.all_reduce_population_count(neq)
            return lax.fori_loop(0, nv, count, jnp.zeros((16,), jnp.int32))

        for r in range(rows_per_w):
            row = wid * rows_per_w + r
            pltpu.sync_copy(gen_hbm.at[row], tok)

            def build_last(i, c):
                t = tok[pl.ds(i * 16, 16)]
                _, last = plsc.scan_count(t)
                plsc.store_scatter(table, [t], i * 16 + lanes, mask=last)
                return c
            lax.fori_loop(0, nv, build_last, 0)

            def flag_non_last(i, c):
                t = tok[pl.ds(i * 16, 16)]
                lp = plsc.load_gather(table, [t])
                dpos[pl.ds(i * 16, 16)] = (
                    lp != i * 16 + lanes).astype(jnp.int32)
                return c
            lax.fori_loop(0, nv, flag_non_last, 0)

            def build_first(i, c):
                j = nv - 1 - i
                t = tok[pl.ds(j * 16, 16)]
                tr = lax.rev(t, (0,))
                _, firstm = plsc.scan_count(tr)
                plsc.store_scatter(table, [tr], j * 16 + (15 - lanes),
                                   mask=firstm)
                return c
            lax.fori_loop(0, nv, build_first, 0)

            def flag_non_first(i, c):
                t = tok[pl.ds(i * 16, 16)]
                fp = plsc.load_gather(table, [t])
                dpos[pl.ds(i * 16, 16)] = dpos[pl.ds(i * 16, 16)] | (
                    fp != i * 16 + lanes).astype(jnp.int32)
                return c
            lax.fori_loop(0, nv, flag_non_first, 0)

            def count_cands(i, ncand):
                f = dpos[pl.ds(i * 16, 16)]
                for k in range(1, NGRAM):
                    f = f & dpos[pl.ds(i * 16 + k, 16)]
                valid = (i * 16 + lanes) < wn
                return ncand + jnp.sum(jnp.where(valid, f, 0))
            ncand = lax.fori_loop(0, nv, count_cands, jnp.int32(0))

            acc = lax.cond(
                ncand == 0,
                lambda: jnp.full((16,), wn, jnp.int32),
                radix_distinct)

            out16[...] = acc
            pltpu.sync_copy(out16, cnt_hbm.at[row])

    return ngram_kernel(gen)


def _sim_temp_body(h_ref, p_ref, w_ref, cnt_ref, out_ref, *, wn):
    dn = (((1,), (1,)), ((), ()))
    h1 = lax.dot_general(h_ref[...], w_ref[...], dn,
                         preferred_element_type=jnp.float32)
    h2 = lax.dot_general(p_ref[...], w_ref[...], dn,
                         preferred_element_type=jnp.float32)
    dot = jnp.sum(h1 * h2, axis=-1)
    n1 = jnp.maximum(jnp.sqrt(jnp.sum(h1 * h1, axis=-1)), 1e-8)
    n2 = jnp.maximum(jnp.sqrt(jnp.sum(h2 * h2, axis=-1)), 1e-8)
    sim_pen = jnp.clip(dot / (n1 * n2), 0.0, None) * PW
    uniq = cnt_ref[...][:, 0].astype(jnp.float32)
    rep = (wn - uniq) / wn
    out_ref[...] = (1.0 + (rep + sim_pen) * 0.5)[None, :]


def _sim_temp_call(hidden, prev, w, cnts, wn):
    b = hidden.shape[0]
    return pl.pallas_call(
        functools.partial(_sim_temp_body, wn=float(wn)),
        out_shape=jax.ShapeDtypeStruct((1, b), jnp.float32),
    )(hidden, prev, w, cnts)


def _bias_body(l_ref, b_ref, o_ref):
    o_ref[...] = l_ref[...] + b_ref[...]


def _bias_call(logits, bias):
    b, v = logits.shape
    blk = 12800
    grid = (v + blk - 1) // blk
    return pl.pallas_call(
        _bias_body,
        grid=(grid,),
        in_specs=[pl.BlockSpec((b, blk), lambda i: (0, i)),
                  pl.BlockSpec((1, blk), lambda i: (0, i))],
        out_specs=pl.BlockSpec((b, blk), lambda i: (0, i)),
        out_shape=jax.ShapeDtypeStruct((b, v), jnp.float32),
    )(logits, bias.reshape(1, v))


def _freq_call(out0, gen):
    b, v = out0.shape
    s = gen.shape[1]
    lanes = 16
    rows_per_w = b // 32
    c = float(np.float32(PW) / np.float32(s))
    mesh = plsc.VectorSubcoreMesh(core_axis_name="c", subcore_axis_name="s")

    @functools.partial(
        pl.kernel,
        out_type=jax.ShapeDtypeStruct((b, v), jnp.float32),
        mesh=mesh,
        compiler_params=_SC_PARAMS,
        scratch_types=[
            pltpu.VMEM((v,), jnp.float32),
            pltpu.VMEM((s,), jnp.int32),
        ],
    )
    def freq_kernel(out0_hbm, gen_hbm, out_hbm, buf, tok):
        wid = lax.axis_index("s") * 2 + lax.axis_index("c")
        for r in range(rows_per_w):
            row = wid * rows_per_w + r
            pltpu.sync_copy(out0_hbm.at[row], buf)
            pltpu.sync_copy(gen_hbm.at[row], tok)

            def body(i, carry):
                idx = tok[pl.ds(i * lanes, lanes)]
                cnt, last = plsc.scan_count(idx)
                plsc.addupdate_scatter(
                    buf, [idx], cnt.astype(jnp.float32) * (-c), mask=last)
                return carry

            lax.fori_loop(0, s // lanes, body, 0)
            pltpu.sync_copy(buf, out_hbm.at[row])

    return freq_kernel(out0, gen)


def kernel(logits, hidden_state, prev_hidden, generated_ids, W, bias):
    b, v = logits.shape
    s = generated_ids.shape[1]
    wn = s - NGRAM + 1

    cnts = _ngram_call(generated_ids, v)
    out0 = _bias_call(logits, bias)
    out = _freq_call(out0, generated_ids)
    temp = _sim_temp_call(hidden_state, prev_hidden, W, cnts, wn)
    return out, temp.reshape(b)


# speedup vs baseline: 8.3639x; 1.0983x over previous
import functools

import jax
import jax.numpy as jnp
import numpy as np
from jax import lax
from jax.experimental import pallas as pl
from jax.experimental.pallas import tpu as pltpu
from jax.experimental.pallas import tpu_sc as plsc

PW = 0.1
NGRAM = 4
_SC_PARAMS = pltpu.CompilerParams(needs_layout_passes=False)


def _ngram_call(gen, v):
    b, s = gen.shape
    wn = s - NGRAM + 1
    nv = s // 16
    padw = s
    sent = s
    nbins = s + 1
    histw = ((nbins + 15) // 16) * 16
    rows_per_w = b // 32
    mesh = plsc.VectorSubcoreMesh(core_axis_name="c", subcore_axis_name="s")

    @functools.partial(
        pl.kernel,
        out_type=jax.ShapeDtypeStruct((b, 16), jnp.int32),
        mesh=mesh,
        compiler_params=_SC_PARAMS,
        scratch_types=[
            pltpu.VMEM((v,), jnp.int32),
            pltpu.VMEM((s,), jnp.int32),
            pltpu.VMEM((s + 16,), jnp.int32),
            pltpu.VMEM((s + 16,), jnp.int32),
            pltpu.VMEM((s + 16,), jnp.int32),
            pltpu.VMEM((s + 16,), jnp.int32),
            pltpu.VMEM((histw,), jnp.int32),
            pltpu.VMEM((16,), jnp.int32),
        ],
    )
    def ngram_kernel(gen_hbm, cnt_hbm, table, tok, tids, dpos, ws_a, ws_b,
                     hist, out16):
        wid = lax.axis_index("s") * 2 + lax.axis_index("c")
        lanes = lax.broadcasted_iota(jnp.int32, (16,), 0)
        dpos[pl.ds(s, 16)] = jnp.zeros((16,), jnp.int32)

        def radix_distinct():
            def build_table(i, c):
                t = tok[pl.ds(i * 16, 16)]
                _, last = plsc.scan_count(t)
                plsc.store_scatter(table, [t], i * 16 + lanes, mask=last)
                return c
            lax.fori_loop(0, nv, build_table, 0)

            def fill_ids(i, c):
                t = tok[pl.ds(i * 16, 16)]
                tids[pl.ds(i * 16, 16)] = plsc.load_gather(table, [t])
                return c
            lax.fori_loop(0, nv, fill_ids, 0)
            tids[pl.ds(s, 16)] = jnp.full((16,), sent, jnp.int32)

            def init_order(i, c):
                vals = i * 16 + lanes
                ws_a[pl.ds(i * 16, 16)] = jnp.where(vals < wn, vals, padw)
                return c
            lax.fori_loop(0, nv, init_order, 0)

            for p in range(NGRAM):
                src, dst = (ws_a, ws_b) if p % 2 == 0 else (ws_b, ws_a)
                koff = NGRAM - 1 - p

                def zero_hist(i, c):
                    hist[pl.ds(i * 16, 16)] = jnp.zeros((16,), jnp.int32)
                    return c
                lax.fori_loop(0, histw // 16, zero_hist, 0)

                def histo(i, c):
                    w = src[pl.ds(i * 16, 16)]
                    d = plsc.load_gather(tids, [w + koff])
                    occ, last = plsc.scan_count(d)
                    plsc.addupdate_scatter(hist, [d], occ, mask=last)
                    return c
                lax.fori_loop(0, nv, histo, 0)

                def excl_scan(i, run):
                    h = hist[pl.ds(i * 16, 16)]
                    csum = plsc.cumsum(h)
                    hist[pl.ds(i * 16, 16)] = csum - h + run
                    return run + jnp.sum(h)
                lax.fori_loop(0, histw // 16, excl_scan, jnp.int32(0))

                def permute(i, c):
                    w = src[pl.ds(i * 16, 16)]
                    d = plsc.load_gather(tids, [w + koff])
                    base = plsc.load_gather(hist, [d])
                    occ, last = plsc.scan_count(d)
                    plsc.store_scatter(dst, [base + occ - 1], w)
                    plsc.addupdate_scatter(hist, [d], occ, mask=last)
                    return c
                lax.fori_loop(0, nv, permute, 0)

            final = ws_a if NGRAM % 2 == 0 else ws_b
            final[pl.ds(s, 16)] = jnp.full((16,), padw, jnp.int32)

            def count(i, acc):
                a = final[pl.ds(i * 16, 16)]
                bb = final[pl.ds(i * 16 + 1, 16)]
                neq = jnp.zeros((16,), jnp.bool_)
                for k in range(NGRAM):
                    da = plsc.load_gather(tids, [a + k])
                    db = plsc.load_gather(tids, [bb + k])
                    neq = neq | (da != db)
                return acc + plsc.all_reduce_population_count(neq)
            return lax.fori_loop(0, nv, count, jnp.zeros((16,), jnp.int32))

        for r in range(rows_per_w):
            row = wid * rows_per_w + r
            pltpu.sync_copy(gen_hbm.at[row], tok)

            def build_last(i, c):
                t = tok[pl.ds(i * 16, 16)]
                _, last = plsc.scan_count(t)
                plsc.store_scatter(table, [t], i * 16 + lanes, mask=last)
                return c
            lax.fori_loop(0, nv, build_last, 0)

            @plsc.parallel_loop(0, nv, unroll=4)
            def flag_non_last(i):
                t = tok[pl.ds(i * 16, 16)]
                lp = plsc.load_gather(table, [t])
                dpos[pl.ds(i * 16, 16)] = (
                    lp != i * 16 + lanes).astype(jnp.int32)

            def build_first(i, c):
                j = nv - 1 - i
                t = tok[pl.ds(j * 16, 16)]
                tr = lax.rev(t, (0,))
                _, firstm = plsc.scan_count(tr)
                plsc.store_scatter(table, [tr], j * 16 + (15 - lanes),
                                   mask=firstm)
                return c
            lax.fori_loop(0, nv, build_first, 0)

            @plsc.parallel_loop(0, nv, unroll=4)
            def flag_non_first(i):
                t = tok[pl.ds(i * 16, 16)]
                fp = plsc.load_gather(table, [t])
                dpos[pl.ds(i * 16, 16)] = dpos[pl.ds(i * 16, 16)] | (
                    fp != i * 16 + lanes).astype(jnp.int32)

            @plsc.parallel_loop(0, nv, unroll=4, carry=jnp.int32(0))
            def ncand(i, acc):
                f = dpos[pl.ds(i * 16, 16)]
                for k in range(1, NGRAM):
                    f = f & dpos[pl.ds(i * 16 + k, 16)]
                valid = (i * 16 + lanes) < wn
                return acc + jnp.sum(jnp.where(valid, f, 0))

            acc = lax.cond(
                ncand == 0,
                lambda: jnp.full((16,), wn, jnp.int32),
                radix_distinct)

            out16[...] = acc
            pltpu.sync_copy(out16, cnt_hbm.at[row])

    return ngram_kernel(gen)


def _sim_temp_body(h_ref, p_ref, w_ref, cnt_ref, out_ref, *, wn):
    dn = (((1,), (1,)), ((), ()))
    h1 = lax.dot_general(h_ref[...], w_ref[...], dn,
                         preferred_element_type=jnp.float32)
    h2 = lax.dot_general(p_ref[...], w_ref[...], dn,
                         preferred_element_type=jnp.float32)
    dot = jnp.sum(h1 * h2, axis=-1)
    n1 = jnp.maximum(jnp.sqrt(jnp.sum(h1 * h1, axis=-1)), 1e-8)
    n2 = jnp.maximum(jnp.sqrt(jnp.sum(h2 * h2, axis=-1)), 1e-8)
    sim_pen = jnp.clip(dot / (n1 * n2), 0.0, None) * PW
    uniq = cnt_ref[...][:, 0].astype(jnp.float32)
    rep = (wn - uniq) / wn
    out_ref[...] = (1.0 + (rep + sim_pen) * 0.5)[None, :]


def _sim_temp_call(hidden, prev, w, cnts, wn):
    b = hidden.shape[0]
    return pl.pallas_call(
        functools.partial(_sim_temp_body, wn=float(wn)),
        out_shape=jax.ShapeDtypeStruct((1, b), jnp.float32),
    )(hidden, prev, w, cnts)


def _bias_body(l_ref, b_ref, o_ref):
    o_ref[...] = l_ref[...] + b_ref[...]


def _bias_call(logits, bias):
    b, v = logits.shape
    blk = 12800
    grid = (v + blk - 1) // blk
    return pl.pallas_call(
        _bias_body,
        grid=(grid,),
        in_specs=[pl.BlockSpec((b, blk), lambda i: (0, i)),
                  pl.BlockSpec((1, blk), lambda i: (0, i))],
        out_specs=pl.BlockSpec((b, blk), lambda i: (0, i)),
        out_shape=jax.ShapeDtypeStruct((b, v), jnp.float32),
    )(logits, bias.reshape(1, v))


def _freq_call(out0, gen):
    b, v = out0.shape
    s = gen.shape[1]
    lanes = 16
    rows_per_w = b // 32
    c = float(np.float32(PW) / np.float32(s))
    mesh = plsc.VectorSubcoreMesh(core_axis_name="c", subcore_axis_name="s")

    @functools.partial(
        pl.kernel,
        out_type=jax.ShapeDtypeStruct((b, v), jnp.float32),
        mesh=mesh,
        compiler_params=_SC_PARAMS,
        scratch_types=[
            pltpu.VMEM((v,), jnp.float32),
            pltpu.VMEM((s,), jnp.int32),
        ],
    )
    def freq_kernel(out0_hbm, gen_hbm, out_hbm, buf, tok):
        wid = lax.axis_index("s") * 2 + lax.axis_index("c")
        for r in range(rows_per_w):
            row = wid * rows_per_w + r
            pltpu.sync_copy(out0_hbm.at[row], buf)
            pltpu.sync_copy(gen_hbm.at[row], tok)

            @plsc.parallel_loop(0, s // lanes, unroll=4)
            def body(i):
                idx = tok[pl.ds(i * lanes, lanes)]
                cnt, last = plsc.scan_count(idx)
                plsc.addupdate_scatter(
                    buf, [idx], cnt.astype(jnp.float32) * (-c), mask=last)
            pltpu.sync_copy(buf, out_hbm.at[row])

    return freq_kernel(out0, gen)


def kernel(logits, hidden_state, prev_hidden, generated_ids, W, bias):
    b, v = logits.shape
    s = generated_ids.shape[1]
    wn = s - NGRAM + 1

    cnts = _ngram_call(generated_ids, v)
    out0 = _bias_call(logits, bias)
    out = _freq_call(out0, generated_ids)
    temp = _sim_temp_call(hidden_state, prev_hidden, W, cnts, wn)
    return out, temp.reshape(b)
